# baseline (speedup 1.0000x reference)
import numpy as np
import jax
import jax.numpy as jnp
from itertools import product as _product
from math import sqrt as _sqrt
from jax.experimental import pallas as pl

_PREC = jax.lax.Precision.HIGHEST

_SSD_CFG = {
    'num_classes': 21, 'input_size': 300,
    'bbox_aspect_num': [4, 6, 6, 6, 4, 4],
    'feature_maps': [38, 19, 10, 5, 3, 1],
    'steps': [8, 16, 32, 64, 100, 300],
    'min_sizes': [30, 60, 111, 162, 213, 264],
    'max_sizes': [60, 111, 162, 213, 264, 315],
    'aspect_ratios': [[2], [2, 3], [2, 3], [2, 3], [2], [2]],
}


def _dbox_const():
    cfg = _SSD_CFG
    mean = []
    for k, f in enumerate(cfg['feature_maps']):
        for i, j in _product(range(f), repeat=2):
            f_k = cfg['input_size'] / cfg['steps'][k]
            cx, cy = (j + 0.5) / f_k, (i + 0.5) / f_k
            s_k = cfg['min_sizes'][k] / cfg['input_size']
            mean += [cx, cy, s_k, s_k]
            s_kb = _sqrt(s_k * (cfg['max_sizes'][k] / cfg['input_size']))
            mean += [cx, cy, s_kb, s_kb]
            for ar in cfg['aspect_ratios'][k]:
                sq = _sqrt(ar)
                mean += [cx, cy, s_k * sq, s_k / sq]
                mean += [cx, cy, s_k / sq, s_k * sq]
    out = np.asarray(mean, dtype=np.float32).reshape(-1, 4)
    return jnp.asarray(np.clip(out, 0.0, 1.0))


def _rup(n, m):
    return (n + m - 1) // m * m


def _conv(x, w, b, *, K, pad, relu, bh, th=None, dil=1, bco=None):
    B, H, W, Cin = x.shape
    Cout = w.shape[3]
    span = dil * (K - 1)
    Ho = H + 2 * pad - span
    Wo = W + 2 * pad - span
    W2 = _rup(W + 2 * pad, 8)
    th = th or bh
    assert bh % th == 0
    Hp = _rup(Ho, bh)
    nb = Hp // bh
    rows = bh + span
    bco = bco if bco is not None else min(Cout, 512)
    assert Cout % bco == 0
    nco = Cout // bco

    xp = jnp.pad(x, ((0, 0), (pad, Hp + span - H - pad),
                     (pad, W2 + span - W - pad), (0, 0)))
    planes = []
    for dx in range(K):
        xs = jax.lax.slice_in_dim(xp, dx * dil, dx * dil + W2, axis=2)
        if nb > 1:
            blk = jnp.concatenate(
                [jax.lax.slice_in_dim(xs, i * bh, i * bh + rows, axis=1)[:, None]
                 for i in range(nb)], axis=1)
        else:
            blk = xs[:, None]
        planes.append(blk[:, :, None])
    xb = jnp.concatenate(planes, axis=2).reshape(B * nb, K * rows * W2, Cin)

    b2 = b.reshape(1, Cout)
    M = bh * W2
    Mt = th * W2
    X = K * rows * W2

    def body(x_ref, w_ref, b_ref, o_ref):
        for t in range(bh // th):
            acc = None
            for dy in range(K):
                for dx in range(K):
                    base = (dx * rows + dy * dil + t * th) * W2
                    lhs = x_ref[0, base:base + Mt, :]
                    d = jax.lax.dot_general(
                        lhs, w_ref[dy, dx], (((1,), (0,)), ((), ())),
                        preferred_element_type=jnp.float32, precision=_PREC)
                    acc = d if acc is None else acc + d
            acc = acc + b_ref[...]
            if relu:
                acc = jnp.maximum(acc, 0.0)
            o_ref[0, t * Mt:(t + 1) * Mt, :] = acc

    w_bytes = K * K * Cin * Cout * 4
    x_bytes = xb.size * 4
    if w_bytes > x_bytes and nco > 1:
        grid = (nco, B * nb)
        sx = pl.BlockSpec((1, X, Cin), lambda c, i: (i, 0, 0))
        sw = pl.BlockSpec((K, K, Cin, bco), lambda c, i: (0, 0, 0, c))
        sb = pl.BlockSpec((1, bco), lambda c, i: (0, c))
        so = pl.BlockSpec((1, M, bco), lambda c, i: (i, 0, c))
    else:
        grid = (B * nb, nco)
        sx = pl.BlockSpec((1, X, Cin), lambda i, c: (i, 0, 0))
        sw = pl.BlockSpec((K, K, Cin, bco), lambda i, c: (0, 0, 0, c))
        sb = pl.BlockSpec((1, bco), lambda i, c: (0, c))
        so = pl.BlockSpec((1, M, bco), lambda i, c: (i, 0, c))

    out = pl.pallas_call(
        body, grid=grid,
        in_specs=[sx, sw, sb], out_specs=so,
        out_shape=jax.ShapeDtypeStruct((B * nb, M, Cout), jnp.float32),
    )(xb, w, b2)
    out = out.reshape(B, Hp, W2, Cout)
    return out[:, :Ho, :Wo, :]


def _conv_im2col(x, w_oihw, b, *, stride, pad, bh, th=None):
    B, H, W, Cin = x.shape
    Ho = (H + 2 * pad - 3) // stride + 1
    xp = jnp.pad(x, ((0, 0), (pad, pad), (pad, pad), (0, 0)))
    cols = []
    for dy in range(3):
        for dx in range(3):
            lim_y = dy + stride * (Ho - 1) + 1
            cols.append(jax.lax.slice(
                xp, (0, dy, dx, 0), (B, lim_y, dx + stride * (Ho - 1) + 1, Cin),
                (1, stride, stride, 1)))
    p = jnp.concatenate(cols, axis=3)
    w2 = jnp.transpose(w_oihw, (2, 3, 1, 0)).reshape(1, 1, 9 * Cin, -1)
    return _conv(p, w2, b, K=1, pad=0, relu=True, bh=bh, th=th)


def _maxpool(x, k, s, lo=0, hi=0):
    return jax.lax.reduce_window(
        x, -jnp.inf, jax.lax.max, (1, k, k, 1), (1, s, s, 1),
        ((0, 0), (lo, hi), (lo, hi), (0, 0)))


def kernel(x, vgg_params, extras_params, loc_params, conf_params, l2norm_weight):
    B = x.shape[0]
    t = lambda w: jnp.transpose(w, (2, 3, 1, 0))
    x = jnp.transpose(x, (0, 2, 3, 1))

    bhs = [(20, 4), (20, 5), (30, 10), (30, 10), (25, 5), (25, 5), (25, 5),
           (20, 4), (20, 4), (20, 4), (20, 10), (20, 10), (20, 10),
           (20, 10), (20, 5)]
    p = iter(zip(vgg_params, bhs))

    def vgg_conv(x, **kw):
        (w, b), (bh, th) = next(p)
        return _conv(x, t(w), b, K=3, pad=1, relu=True, bh=bh, th=th, **kw)

    (w0, b0), (bh0, th0) = next(p)
    x = _conv_im2col(x, w0, b0, stride=1, pad=1, bh=bh0, th=th0)
    x = vgg_conv(x)
    x = _maxpool(x, 2, 2)
    x = vgg_conv(x)
    x = vgg_conv(x)
    x = _maxpool(x, 2, 2)
    x = vgg_conv(x)
    x = vgg_conv(x)
    x = vgg_conv(x)
    x = _maxpool(x, 2, 2, 0, 1)
    x = vgg_conv(x)
    x = vgg_conv(x)
    x = vgg_conv(x)
    norm = jnp.sqrt(jnp.sum(x * x, axis=3, keepdims=True)) + 1e-10
    s1 = (x / norm) * l2norm_weight[None, None, None, :]
    sources = [s1]
    x = _maxpool(x, 2, 2)
    x = vgg_conv(x)
    x = vgg_conv(x)
    x = vgg_conv(x)
    x = _maxpool(x, 3, 1, 1, 1)
    (w, b), (bh, th) = next(p)
    x = _conv(x, t(w), b, K=3, pad=6, dil=6, relu=True, bh=bh, th=th)
    (w, b), (bh, th) = next(p)
    x = _conv(x, t(w), b, K=1, pad=0, relu=True, bh=bh, th=th)
    sources.append(x)

    e = extras_params
    x = _conv(x, t(e[0][0]), e[0][1], K=1, pad=0, relu=True, bh=20, th=5)
    x = _conv_im2col(x, e[1][0], e[1][1], stride=2, pad=1, bh=10, th=5)
    sources.append(x)
    x = _conv(x, t(e[2][0]), e[2][1], K=1, pad=0, relu=True, bh=10, th=10)
    x = _conv_im2col(x, e[3][0], e[3][1], stride=2, pad=1, bh=5)
    sources.append(x)
    x = _conv(x, t(e[4][0]), e[4][1], K=1, pad=0, relu=True, bh=5)
    x = _conv(x, t(e[5][0]), e[5][1], K=3, pad=0, relu=True, bh=3)
    sources.append(x)
    x = _conv(x, t(e[6][0]), e[6][1], K=1, pad=0, relu=True, bh=3)
    x = _conv(x, t(e[7][0]), e[7][1], K=3, pad=0, relu=True, bh=1)
    sources.append(x)

    head_bh = [(20, 4), (20, 5), (10, 10), (5, 5), (3, 3), (1, 1)]
    locs, confs = [], []
    for s, (lw, lb), (cw, cb), (bh, th) in zip(sources, loc_params, conf_params, head_bh):
        l = _conv(s, t(lw), lb, K=3, pad=1, relu=False, bh=bh, th=th)
        c = _conv(s, t(cw), cb, K=3, pad=1, relu=False, bh=bh, th=th)
        locs.append(l.reshape(B, -1))
        confs.append(c.reshape(B, -1))
    loc = jnp.concatenate(locs, axis=1).reshape(B, -1, 4)
    conf = jnp.concatenate(confs, axis=1).reshape(B, -1, _SSD_CFG['num_classes'])
    return loc, conf, _dbox_const()


# speedup vs baseline: 1.0291x; 1.0291x over previous
import numpy as np
import jax
import jax.numpy as jnp
from itertools import product as _product
from math import sqrt as _sqrt
from jax.experimental import pallas as pl

_SSD_CFG = {
    'num_classes': 21, 'input_size': 300,
    'bbox_aspect_num': [4, 6, 6, 6, 4, 4],
    'feature_maps': [38, 19, 10, 5, 3, 1],
    'steps': [8, 16, 32, 64, 100, 300],
    'min_sizes': [30, 60, 111, 162, 213, 264],
    'max_sizes': [60, 111, 162, 213, 264, 315],
    'aspect_ratios': [[2], [2, 3], [2, 3], [2, 3], [2], [2]],
}


def _dbox_const():
    cfg = _SSD_CFG
    mean = []
    for k, f in enumerate(cfg['feature_maps']):
        for i, j in _product(range(f), repeat=2):
            f_k = cfg['input_size'] / cfg['steps'][k]
            cx, cy = (j + 0.5) / f_k, (i + 0.5) / f_k
            s_k = cfg['min_sizes'][k] / cfg['input_size']
            mean += [cx, cy, s_k, s_k]
            s_kb = _sqrt(s_k * (cfg['max_sizes'][k] / cfg['input_size']))
            mean += [cx, cy, s_kb, s_kb]
            for ar in cfg['aspect_ratios'][k]:
                sq = _sqrt(ar)
                mean += [cx, cy, s_k * sq, s_k / sq]
                mean += [cx, cy, s_k / sq, s_k * sq]
    out = np.asarray(mean, dtype=np.float32).reshape(-1, 4)
    return jnp.asarray(np.clip(out, 0.0, 1.0))


def _rup(n, m):
    return (n + m - 1) // m * m


def _conv(x, w, b, *, K, pad, relu, bh, th=None, dil=1, bco=None):
    B, H, W, Cin = x.shape
    Cout = w.shape[3]
    span = dil * (K - 1)
    Ho = H + 2 * pad - span
    Wo = W + 2 * pad - span
    W2 = _rup(W + 2 * pad, 16)
    th = th or bh
    assert bh % th == 0
    Hp = _rup(Ho, bh)
    nb = Hp // bh
    rows = bh + span
    bco = bco if bco is not None else min(Cout, 512)
    assert Cout % bco == 0
    nco = Cout // bco

    xp = jnp.pad(x, ((0, 0), (pad, Hp + span - H - pad),
                     (pad, W2 + span - W - pad), (0, 0)))
    planes = []
    for dx in range(K):
        xs = jax.lax.slice_in_dim(xp, dx * dil, dx * dil + W2, axis=2)
        if nb > 1:
            blk = jnp.concatenate(
                [jax.lax.slice_in_dim(xs, i * bh, i * bh + rows, axis=1)[:, None]
                 for i in range(nb)], axis=1)
        else:
            blk = xs[:, None]
        planes.append(blk[:, :, None])
    xb = jnp.concatenate(planes, axis=2).reshape(B * nb, K * rows * W2, Cin)

    b2 = b.reshape(1, Cout)
    M = bh * W2
    Mt = th * W2
    X = K * rows * W2

    f32, bf16 = jnp.float32, jnp.bfloat16
    xhi = xb.astype(bf16)
    xlo = (xb - xhi.astype(f32)).astype(bf16)
    whi = w.astype(bf16)
    wlo = (w - whi.astype(f32)).astype(bf16)

    def _dot(a, bm):
        return jax.lax.dot_general(
            a, bm, (((1,), (0,)), ((), ())),
            preferred_element_type=f32, precision=jax.lax.Precision.DEFAULT)

    def body(xh_ref, xl_ref, wh_ref, wl_ref, b_ref, o_ref):
        for t in range(bh // th):
            acc = None
            for dy in range(K):
                for dx in range(K):
                    base = (dx * rows + dy * dil + t * th) * W2
                    ah = xh_ref[0, base:base + Mt, :]
                    al = xl_ref[0, base:base + Mt, :]
                    wh = wh_ref[dy, dx]
                    wl = wl_ref[dy, dx]
                    d = _dot(ah, wh) + (_dot(ah, wl) + _dot(al, wh))
                    acc = d if acc is None else acc + d
            acc = acc + b_ref[...]
            if relu:
                acc = jnp.maximum(acc, 0.0)
            o_ref[0, t * Mt:(t + 1) * Mt, :] = acc

    w_bytes = K * K * Cin * Cout * 4
    x_bytes = xb.size * 4
    if w_bytes > x_bytes and nco > 1:
        grid = (nco, B * nb)
        gx, gw = (lambda c, i: (i, 0, 0)), (lambda c, i: (0, 0, 0, c))
        gb, go = (lambda c, i: (0, c)), (lambda c, i: (i, 0, c))
    else:
        grid = (B * nb, nco)
        gx, gw = (lambda i, c: (i, 0, 0)), (lambda i, c: (0, 0, 0, c))
        gb, go = (lambda i, c: (0, c)), (lambda i, c: (i, 0, c))
    sx = pl.BlockSpec((1, X, Cin), gx)
    sw = pl.BlockSpec((K, K, Cin, bco), gw)
    sb = pl.BlockSpec((1, bco), gb)
    so = pl.BlockSpec((1, M, bco), go)

    out = pl.pallas_call(
        body, grid=grid,
        in_specs=[sx, sx, sw, sw, sb], out_specs=so,
        out_shape=jax.ShapeDtypeStruct((B * nb, M, Cout), jnp.float32),
    )(xhi, xlo, whi, wlo, b2)
    out = out.reshape(B, Hp, W2, Cout)
    return out[:, :Ho, :Wo, :]


def _conv_im2col(x, w_oihw, b, *, stride, pad, bh, th=None):
    B, H, W, Cin = x.shape
    Ho = (H + 2 * pad - 3) // stride + 1
    xp = jnp.pad(x, ((0, 0), (pad, pad), (pad, pad), (0, 0)))
    cols = []
    for dy in range(3):
        for dx in range(3):
            lim_y = dy + stride * (Ho - 1) + 1
            cols.append(jax.lax.slice(
                xp, (0, dy, dx, 0), (B, lim_y, dx + stride * (Ho - 1) + 1, Cin),
                (1, stride, stride, 1)))
    p = jnp.concatenate(cols, axis=3)
    w2 = jnp.transpose(w_oihw, (2, 3, 1, 0)).reshape(1, 1, 9 * Cin, -1)
    return _conv(p, w2, b, K=1, pad=0, relu=True, bh=bh, th=th)


def _maxpool(x, k, s, lo=0, hi=0):
    return jax.lax.reduce_window(
        x, -jnp.inf, jax.lax.max, (1, k, k, 1), (1, s, s, 1),
        ((0, 0), (lo, hi), (lo, hi), (0, 0)))


def kernel(x, vgg_params, extras_params, loc_params, conf_params, l2norm_weight):
    B = x.shape[0]
    t = lambda w: jnp.transpose(w, (2, 3, 1, 0))
    x = jnp.transpose(x, (0, 2, 3, 1))

    bhs = [(20, 4), (20, 5), (30, 10), (30, 10), (25, 5), (25, 5), (25, 5),
           (20, 4), (20, 4), (20, 4), (20, 10), (20, 10), (20, 10),
           (20, 10), (20, 5)]
    p = iter(zip(vgg_params, bhs))

    def vgg_conv(x, **kw):
        (w, b), (bh, th) = next(p)
        return _conv(x, t(w), b, K=3, pad=1, relu=True, bh=bh, th=th, **kw)

    (w0, b0), (bh0, th0) = next(p)
    x = _conv_im2col(x, w0, b0, stride=1, pad=1, bh=bh0, th=th0)
    x = vgg_conv(x)
    x = _maxpool(x, 2, 2)
    x = vgg_conv(x)
    x = vgg_conv(x)
    x = _maxpool(x, 2, 2)
    x = vgg_conv(x)
    x = vgg_conv(x)
    x = vgg_conv(x)
    x = _maxpool(x, 2, 2, 0, 1)
    x = vgg_conv(x)
    x = vgg_conv(x)
    x = vgg_conv(x)
    norm = jnp.sqrt(jnp.sum(x * x, axis=3, keepdims=True)) + 1e-10
    s1 = (x / norm) * l2norm_weight[None, None, None, :]
    sources = [s1]
    x = _maxpool(x, 2, 2)
    x = vgg_conv(x)
    x = vgg_conv(x)
    x = vgg_conv(x)
    x = _maxpool(x, 3, 1, 1, 1)
    (w, b), (bh, th) = next(p)
    x = _conv(x, t(w), b, K=3, pad=6, dil=6, relu=True, bh=bh, th=th)
    (w, b), (bh, th) = next(p)
    x = _conv(x, t(w), b, K=1, pad=0, relu=True, bh=bh, th=th)
    sources.append(x)

    e = extras_params
    x = _conv(x, t(e[0][0]), e[0][1], K=1, pad=0, relu=True, bh=20, th=5)
    x = _conv_im2col(x, e[1][0], e[1][1], stride=2, pad=1, bh=10, th=5)
    sources.append(x)
    x = _conv(x, t(e[2][0]), e[2][1], K=1, pad=0, relu=True, bh=10, th=10)
    x = _conv_im2col(x, e[3][0], e[3][1], stride=2, pad=1, bh=5)
    sources.append(x)
    x = _conv(x, t(e[4][0]), e[4][1], K=1, pad=0, relu=True, bh=5)
    x = _conv(x, t(e[5][0]), e[5][1], K=3, pad=0, relu=True, bh=3)
    sources.append(x)
    x = _conv(x, t(e[6][0]), e[6][1], K=1, pad=0, relu=True, bh=3)
    x = _conv(x, t(e[7][0]), e[7][1], K=3, pad=0, relu=True, bh=1)
    sources.append(x)

    head_bh = [(20, 4), (20, 5), (10, 10), (5, 5), (3, 3), (1, 1)]
    locs, confs = [], []
    for s, (lw, lb), (cw, cb), (bh, th) in zip(sources, loc_params, conf_params, head_bh):
        l = _conv(s, t(lw), lb, K=3, pad=1, relu=False, bh=bh, th=th)
        c = _conv(s, t(cw), cb, K=3, pad=1, relu=False, bh=bh, th=th)
        locs.append(l.reshape(B, -1))
        confs.append(c.reshape(B, -1))
    loc = jnp.concatenate(locs, axis=1).reshape(B, -1, 4)
    conf = jnp.concatenate(confs, axis=1).reshape(B, -1, _SSD_CFG['num_classes'])
    return loc, conf, _dbox_const()


# speedup vs baseline: 1.2259x; 1.1912x over previous
import numpy as np
import jax
import jax.numpy as jnp
from itertools import product as _product
from math import sqrt as _sqrt
from jax.experimental import pallas as pl
from jax.experimental.pallas import tpu as pltpu

_SSD_CFG = {
    'num_classes': 21, 'input_size': 300,
    'bbox_aspect_num': [4, 6, 6, 6, 4, 4],
    'feature_maps': [38, 19, 10, 5, 3, 1],
    'steps': [8, 16, 32, 64, 100, 300],
    'min_sizes': [30, 60, 111, 162, 213, 264],
    'max_sizes': [60, 111, 162, 213, 264, 315],
    'aspect_ratios': [[2], [2, 3], [2, 3], [2, 3], [2], [2]],
}


def _dbox_const():
    cfg = _SSD_CFG
    mean = []
    for k, f in enumerate(cfg['feature_maps']):
        for i, j in _product(range(f), repeat=2):
            f_k = cfg['input_size'] / cfg['steps'][k]
            cx, cy = (j + 0.5) / f_k, (i + 0.5) / f_k
            s_k = cfg['min_sizes'][k] / cfg['input_size']
            mean += [cx, cy, s_k, s_k]
            s_kb = _sqrt(s_k * (cfg['max_sizes'][k] / cfg['input_size']))
            mean += [cx, cy, s_kb, s_kb]
            for ar in cfg['aspect_ratios'][k]:
                sq = _sqrt(ar)
                mean += [cx, cy, s_k * sq, s_k / sq]
                mean += [cx, cy, s_k / sq, s_k * sq]
    out = np.asarray(mean, dtype=np.float32).reshape(-1, 4)
    return jnp.asarray(np.clip(out, 0.0, 1.0))


def _rup(n, m):
    return (n + m - 1) // m * m


def _conv(x, w, b, *, K, pad, relu, bh, th=None, dil=1, bco=None):
    B, H, W, Cin = x.shape
    Cout = w.shape[3]
    span = dil * (K - 1)
    Ho = H + 2 * pad - span
    Wo = W + 2 * pad - span
    W2 = _rup(W + 2 * pad, 16)
    th = th or bh
    assert bh % th == 0
    Hp = _rup(Ho, bh)
    nb = Hp // bh
    rows = bh + span
    bco = bco if bco is not None else min(Cout, 512)
    assert Cout % bco == 0
    nco = Cout // bco

    xp = jnp.pad(x, ((0, 0), (pad, Hp + span + 1 - H - pad),
                     (pad, W2 - W - pad), (0, 0)))
    if nb > 1:
        xb = jnp.concatenate(
            [jax.lax.slice_in_dim(xp, i * bh, i * bh + rows + 1, axis=1)[:, None]
             for i in range(nb)], axis=1)
        xb = xb.reshape(B * nb, (rows + 1) * W2, Cin)
    else:
        xb = xp.reshape(B, (rows + 1) * W2, Cin)

    b2 = b.reshape(1, Cout)
    M = bh * W2
    Mt = th * W2
    L = rows * W2
    X = (rows + 1) * W2

    f32, bf16 = jnp.float32, jnp.bfloat16
    xhi = xb.astype(bf16)
    xlo = (xb - xhi.astype(f32)).astype(bf16)
    whi = w.astype(bf16)
    wlo = (w - whi.astype(f32)).astype(bf16)

    def _dot(a, bm):
        return jax.lax.dot_general(
            a, bm, (((1,), (0,)), ((), ())),
            preferred_element_type=f32, precision=jax.lax.Precision.DEFAULT)

    def body(xh_ref, xl_ref, wh_ref, wl_ref, b_ref, o_ref, *scr):
        if K > 1:
            sh_ref, sl_ref = scr
            for dx in range(1, K):
                off = dx * dil
                for r0 in range(0, rows, th):
                    r1 = min(r0 + th, rows)
                    a0, a1 = r0 * W2, r1 * W2
                    sh_ref[dx - 1, a0:a1, :] = xh_ref[0, off + a0:off + a1, :]
                    sl_ref[dx - 1, a0:a1, :] = xl_ref[0, off + a0:off + a1, :]

        def _lhs(ref0, sref, dy, dx, t):
            s = (dy * dil + t * th) * W2
            if dx == 0:
                return ref0[0, s:s + Mt, :]
            return sref[dx - 1, s:s + Mt, :]

        for t in range(bh // th):
            acc = None
            for dy in range(K):
                for dx in range(K):
                    ah = _lhs(xh_ref, scr[0] if K > 1 else None, dy, dx, t)
                    al = _lhs(xl_ref, scr[1] if K > 1 else None, dy, dx, t)
                    wh = wh_ref[dy, dx]
                    wl = wl_ref[dy, dx]
                    d = _dot(ah, wh) + (_dot(ah, wl) + _dot(al, wh))
                    acc = d if acc is None else acc + d
            acc = acc + b_ref[...]
            if relu:
                acc = jnp.maximum(acc, 0.0)
            o_ref[0, t * Mt:(t + 1) * Mt, :] = acc

    w_bytes = K * K * Cin * Cout * 4
    x_bytes = xb.size * 4
    if w_bytes > x_bytes and nco > 1:
        grid = (nco, B * nb)
        gx, gw = (lambda c, i: (i, 0, 0)), (lambda c, i: (0, 0, 0, c))
        gb, go = (lambda c, i: (0, c)), (lambda c, i: (i, 0, c))
    else:
        grid = (B * nb, nco)
        gx, gw = (lambda i, c: (i, 0, 0)), (lambda i, c: (0, 0, 0, c))
        gb, go = (lambda i, c: (0, c)), (lambda i, c: (i, 0, c))
    sx = pl.BlockSpec((1, X, Cin), gx)
    sw = pl.BlockSpec((K, K, Cin, bco), gw)
    sb = pl.BlockSpec((1, bco), gb)
    so = pl.BlockSpec((1, M, bco), go)

    scratch = ([pltpu.VMEM((K - 1, L, Cin), bf16)] * 2) if K > 1 else []
    out = pl.pallas_call(
        body, grid=grid,
        in_specs=[sx, sx, sw, sw, sb], out_specs=so,
        scratch_shapes=scratch,
        out_shape=jax.ShapeDtypeStruct((B * nb, M, Cout), jnp.float32),
    )(xhi, xlo, whi, wlo, b2)
    out = out.reshape(B, Hp, W2, Cout)
    return out[:, :Ho, :Wo, :]


def _conv3_s2d(x, w_hwio, b, *, bh, th, pool):
    B, H, W, C = x.shape
    Co = w_hwio.shape[3]
    xq = jnp.pad(x, ((0, 0), (1, 3), (1, 3), (0, 0)))
    Hq, Wq = H + 4, W + 4
    S = xq.reshape(B, Hq // 2, 2, Wq // 2, 2, C)
    S = S.transpose(0, 1, 3, 2, 4, 5).reshape(B, Hq // 2, Wq // 2, 4 * C)
    WT = jnp.zeros((2, 2, 2, 2, C, 2, 2, Co), w_hwio.dtype)
    for ty in range(2):
        for tx in range(2):
            for qy2 in range(2):
                for qx2 in range(2):
                    for qy in range(2):
                        for qx in range(2):
                            dy = 2 * ty + qy2 - qy
                            dx = 2 * tx + qx2 - qx
                            if 0 <= dy < 3 and 0 <= dx < 3:
                                WT = WT.at[ty, tx, qy2, qx2, :, qy, qx, :].set(
                                    w_hwio[dy, dx])
    WT = WT.reshape(2, 2, 4 * C, 4 * Co)
    y = _conv(S, WT, jnp.tile(b, 4), K=2, pad=0, relu=True, bh=bh, th=th)
    Ho2, Wo2 = H // 2, W // 2
    y = y[:, :Ho2, :Wo2].reshape(B, Ho2, Wo2, 2, 2, Co)
    if pool:
        return jnp.max(y, axis=(3, 4))
    y = y.transpose(0, 1, 3, 2, 4, 5).reshape(B, H, W, Co)
    return y


def _conv_im2col(x, w_oihw, b, *, stride, pad, bh, th=None):
    B, H, W, Cin = x.shape
    Ho = (H + 2 * pad - 3) // stride + 1
    xp = jnp.pad(x, ((0, 0), (pad, pad), (pad, pad), (0, 0)))
    cols = []
    for dy in range(3):
        for dx in range(3):
            lim_y = dy + stride * (Ho - 1) + 1
            cols.append(jax.lax.slice(
                xp, (0, dy, dx, 0), (B, lim_y, dx + stride * (Ho - 1) + 1, Cin),
                (1, stride, stride, 1)))
    p = jnp.concatenate(cols, axis=3)
    w2 = jnp.transpose(w_oihw, (2, 3, 1, 0)).reshape(1, 1, 9 * Cin, -1)
    return _conv(p, w2, b, K=1, pad=0, relu=True, bh=bh, th=th)


def _maxpool(x, k, s, lo=0, hi=0):
    return jax.lax.reduce_window(
        x, -jnp.inf, jax.lax.max, (1, k, k, 1), (1, s, s, 1),
        ((0, 0), (lo, hi), (lo, hi), (0, 0)))


def kernel(x, vgg_params, extras_params, loc_params, conf_params, l2norm_weight):
    B = x.shape[0]
    t = lambda w: jnp.transpose(w, (2, 3, 1, 0))
    x = jnp.transpose(x, (0, 2, 3, 1))

    v = vgg_params
    x = _conv_im2col(x, v[0][0], v[0][1], stride=1, pad=1, bh=20, th=4)
    x = _conv3_s2d(x, t(v[1][0]), v[1][1], bh=16, th=4, pool=True)
    x = _conv3_s2d(x, t(v[2][0]), v[2][1], bh=20, th=4, pool=False)
    x = _conv3_s2d(x, t(v[3][0]), v[3][1], bh=20, th=4, pool=True)
    bhs = [(25, 5), (25, 5), (25, 5),
           (20, 4), (20, 4), (20, 4), (20, 10), (20, 10), (20, 10),
           (20, 10), (20, 5)]
    p = iter(zip(vgg_params[4:], bhs))

    def vgg_conv(x, **kw):
        (w, b), (bh, th) = next(p)
        return _conv(x, t(w), b, K=3, pad=1, relu=True, bh=bh, th=th, **kw)

    x = vgg_conv(x)
    x = vgg_conv(x)
    x = vgg_conv(x)
    x = _maxpool(x, 2, 2, 0, 1)
    x = vgg_conv(x)
    x = vgg_conv(x)
    x = vgg_conv(x)
    norm = jnp.sqrt(jnp.sum(x * x, axis=3, keepdims=True)) + 1e-10
    s1 = (x / norm) * l2norm_weight[None, None, None, :]
    sources = [s1]
    x = _maxpool(x, 2, 2)
    x = vgg_conv(x)
    x = vgg_conv(x)
    x = vgg_conv(x)
    x = _maxpool(x, 3, 1, 1, 1)
    (w, b), (bh, th) = next(p)
    x = _conv(x, t(w), b, K=3, pad=6, dil=6, relu=True, bh=bh, th=th)
    (w, b), (bh, th) = next(p)
    x = _conv(x, t(w), b, K=1, pad=0, relu=True, bh=bh, th=th)
    sources.append(x)

    e = extras_params
    x = _conv(x, t(e[0][0]), e[0][1], K=1, pad=0, relu=True, bh=20, th=5)
    x = _conv_im2col(x, e[1][0], e[1][1], stride=2, pad=1, bh=10, th=5)
    sources.append(x)
    x = _conv(x, t(e[2][0]), e[2][1], K=1, pad=0, relu=True, bh=10, th=10)
    x = _conv_im2col(x, e[3][0], e[3][1], stride=2, pad=1, bh=5)
    sources.append(x)
    x = _conv(x, t(e[4][0]), e[4][1], K=1, pad=0, relu=True, bh=5)
    x = _conv(x, t(e[5][0]), e[5][1], K=3, pad=0, relu=True, bh=3)
    sources.append(x)
    x = _conv(x, t(e[6][0]), e[6][1], K=1, pad=0, relu=True, bh=3)
    x = _conv(x, t(e[7][0]), e[7][1], K=3, pad=0, relu=True, bh=1)
    sources.append(x)

    head_bh = [(20, 4), (20, 5), (10, 10), (5, 5), (3, 3), (1, 1)]
    locs, confs = [], []
    for s, (lw, lb), (cw, cb), (bh, th) in zip(sources, loc_params, conf_params, head_bh):
        l = _conv(s, t(lw), lb, K=3, pad=1, relu=False, bh=bh, th=th)
        c = _conv(s, t(cw), cb, K=3, pad=1, relu=False, bh=bh, th=th)
        locs.append(l.reshape(B, -1))
        confs.append(c.reshape(B, -1))
    loc = jnp.concatenate(locs, axis=1).reshape(B, -1, 4)
    conf = jnp.concatenate(confs, axis=1).reshape(B, -1, _SSD_CFG['num_classes'])
    return loc, conf, _dbox_const()
